# baseline (device time: 628609 ns/iter reference)
import functools

import jax
import jax.numpy as jnp
from jax import lax
from jax.experimental import pallas as pl
from jax.experimental.pallas import tpu as pltpu

N_DEV = 8
CH = 512


def kernel(A, B):
    m = A.shape[0]
    n = B.shape[1]

    GROUP_COLS = ((0, 1408), (1408, 1408), (2816, 1280))

    P = jnp.dot(A, B, preferred_element_type=jnp.float32)

    def body(p_ref, out_ref, rsbuf, vacc, ptile,
             send_sems, recv_sems, add_sems, cp_sem):
        my = lax.axis_index("i")
        z = my // 4
        jj = my - 4 * z
        y = jj // 2
        p = jj - 2 * y
        x = y + p - 2 * y * p
        px = 4 * z + jj + 1 - 2 * p
        py = 4 * z + 3 - jj
        pz = my + 4 - 8 * z

        barrier_sem = pltpu.get_barrier_semaphore()
        for nbr in (px, py, pz):
            pl.semaphore_signal(
                barrier_sem, inc=1,
                device_id=(nbr,), device_id_type=pl.DeviceIdType.MESH,
            )
        pl.semaphore_wait(barrier_sem, 3)

        bits = {"x": x, "y": y, "z": z}
        parts = {"x": px, "y": py, "z": pz}
        orders = (("x", "y", "z"), ("y", "z", "x"), ("z", "x", "y"))
        meta = []
        for g, (col0, w) in enumerate(GROUP_COLS):
            a1, a2, a3 = orders[g]
            b1, b2, b3 = bits[a1], bits[a2], bits[a3]
            k1 = 2048 * b1
            k2l = 1024 * b2
            k3l = k2l + 512 * b3
            meta.append(dict(
                col0=col0, w=w,
                k1=k1, s1=2048 - k1,
                k2l=k2l, s2l=1024 - 1024 * b2,
                k3l=k3l, s3l=k2l + 512 - 512 * b3,
                part=(parts[a1], parts[a2], parts[a3]),
            ))

        rdmas = []
        for g, mt in enumerate(meta):
            cols = pl.ds(mt["col0"], mt["w"])
            r = pltpu.make_async_remote_copy(
                src_ref=p_ref.at[pl.ds(mt["s1"], 2048), cols],
                dst_ref=vacc.at[:, cols],
                send_sem=send_sems.at[0, g],
                recv_sem=recv_sems.at[0, g],
                device_id=(mt["part"][0],),
                device_id_type=pl.DeviceIdType.MESH,
            )
            r.start()
            rdmas.append(r)

        tiles = [(g, t) for g in range(len(meta)) for t in range(2048 // CH)]

        def s1_load(ti):
            g, t = tiles[ti]
            mt = meta[g]
            cols = pl.ds(mt["col0"], mt["w"])
            return pltpu.make_async_copy(
                p_ref.at[pl.ds(mt["k1"] + t * CH, CH), cols],
                ptile.at[ti % 2, :, cols], add_sems.at[ti % 2])

        def s1_start(ti):
            g, t = tiles[ti]
            if t == 0:
                rdmas[g].wait()
            s1_load(ti).start()

        s1_start(0)
        for ti in range(len(tiles)):
            if ti + 1 < len(tiles):
                s1_start(ti + 1)
            g, t = tiles[ti]
            mt = meta[g]
            cols = pl.ds(mt["col0"], mt["w"])
            rows = pl.ds(t * CH, CH)
            s1_load(ti).wait()
            vacc[rows, cols] = vacc[rows, cols] + ptile[ti % 2, :, cols]

        for s in (1, 2):
            L = 2048 >> s
            slot = 0 if s == 1 else 1024
            rdmas = []
            for g, mt in enumerate(meta):
                cols = pl.ds(mt["col0"], mt["w"])
                sendb = mt["s2l"] if s == 1 else mt["s3l"]
                r = pltpu.make_async_remote_copy(
                    src_ref=vacc.at[pl.ds(sendb, L), cols],
                    dst_ref=rsbuf.at[pl.ds(slot, L), cols],
                    send_sem=send_sems.at[s, g],
                    recv_sem=recv_sems.at[s, g],
                    device_id=(mt["part"][s],),
                    device_id_type=pl.DeviceIdType.MESH,
                )
                r.start()
                rdmas.append(r)
            stiles = [(g, t) for g in range(len(meta)) for t in range(L // CH)]

            def rs_load(ti, stiles=stiles, slot=slot):
                g, t = stiles[ti]
                mt = meta[g]
                cols = pl.ds(mt["col0"], mt["w"])
                return pltpu.make_async_copy(
                    rsbuf.at[pl.ds(slot + t * CH, CH), cols],
                    ptile.at[ti % 2, :, cols], add_sems.at[ti % 2])

            def rs_start(ti, stiles=stiles, rdmas=rdmas, rs_load=rs_load):
                g, t = stiles[ti]
                if t == 0:
                    rdmas[g].wait()
                rs_load(ti).start()

            rs_start(0)
            for ti in range(len(stiles)):
                if ti + 1 < len(stiles):
                    rs_start(ti + 1)
                g, t = stiles[ti]
                mt = meta[g]
                cols = pl.ds(mt["col0"], mt["w"])
                keepb = mt["k2l"] if s == 1 else mt["k3l"]
                rows = pl.ds(keepb + t * CH, CH)
                rs_load(ti).wait()
                vacc[rows, cols] = vacc[rows, cols] + ptile[ti % 2, :, cols]

        for mt in meta:
            cols = pl.ds(mt["col0"], mt["w"])
            cp = pltpu.make_async_copy(
                vacc.at[pl.ds(mt["k3l"], CH), cols],
                out_ref.at[pl.ds(mt["k1"] + mt["k3l"], CH), cols],
                cp_sem)
            cp.start()
            cp.wait()

        for si in range(3):
            s = 3 + si
            L = 512 << si
            rdmas = []
            for g, mt in enumerate(meta):
                cols = pl.ds(mt["col0"], mt["w"])
                if si == 0:
                    base = mt["k1"] + mt["k3l"]
                elif si == 1:
                    base = mt["k1"] + mt["k2l"]
                else:
                    base = mt["k1"]
                rows = pl.ds(base, L)
                r = pltpu.make_async_remote_copy(
                    src_ref=out_ref.at[rows, cols],
                    dst_ref=out_ref.at[rows, cols],
                    send_sem=send_sems.at[s, g],
                    recv_sem=recv_sems.at[s, g],
                    device_id=(mt["part"][2 - si],),
                    device_id_type=pl.DeviceIdType.MESH,
                )
                r.start()
                rdmas.append(r)
            for r in rdmas:
                r.wait()

        @functools.partial(
            pl.run_scoped, second_barrier=pltpu.SemaphoreType.REGULAR)
        def _(second_barrier):
            for nbr in (px, py, pz):
                pl.semaphore_signal(
                    second_barrier, inc=1,
                    device_id=(nbr,), device_id_type=pl.DeviceIdType.MESH,
                )
            pl.semaphore_wait(second_barrier, 3)

    out, _ = pl.pallas_call(
        body,
        out_shape=[
            jax.ShapeDtypeStruct((m, n), jnp.float32),
            jax.ShapeDtypeStruct((1536, n), jnp.float32),
        ],
        in_specs=[pl.BlockSpec(memory_space=pl.ANY)],
        out_specs=[
            pl.BlockSpec(memory_space=pl.ANY),
            pl.BlockSpec(memory_space=pl.ANY),
        ],
        scratch_shapes=[
            pltpu.VMEM((2048, n), jnp.float32),
            pltpu.VMEM((2, CH, n), jnp.float32),
            pltpu.SemaphoreType.DMA((6, 3)),
            pltpu.SemaphoreType.DMA((6, 3)),
            pltpu.SemaphoreType.DMA((2,)),
            pltpu.SemaphoreType.DMA,
        ],
        compiler_params=pltpu.CompilerParams(
            collective_id=0,
            vmem_limit_bytes=63 * 1024 * 1024,
        ),
    )(P)
    return out


# device time: 619641 ns/iter; 1.0145x vs baseline; 1.0145x over previous
import functools

import jax
import jax.numpy as jnp
from jax import lax
from jax.experimental import pallas as pl
from jax.experimental.pallas import tpu as pltpu

N_DEV = 8
CH = 512


def kernel(A, B):
    m = A.shape[0]
    n = B.shape[1]

    GROUP_COLS = ((0, 1408), (1408, 1408), (2816, 1280))

    P = jnp.dot(A, B, preferred_element_type=jnp.float32)

    def body(p_ref, out_ref, rsbuf, vacc, ptile,
             send_sems, recv_sems, add_sems, cp_sem,
             ag_send_sems, ag_recv_sems):
        my = lax.axis_index("i")
        z = my // 4
        jj = my - 4 * z
        y = jj // 2
        p = jj - 2 * y
        x = y + p - 2 * y * p
        px = 4 * z + jj + 1 - 2 * p
        py = 4 * z + 3 - jj
        pz = my + 4 - 8 * z

        barrier_sem = pltpu.get_barrier_semaphore()
        for nbr in (px, py, pz):
            pl.semaphore_signal(
                barrier_sem, inc=1,
                device_id=(nbr,), device_id_type=pl.DeviceIdType.MESH,
            )
        pl.semaphore_wait(barrier_sem, 3)

        bits = {"x": x, "y": y, "z": z}
        parts = {"x": px, "y": py, "z": pz}
        orders = (("x", "y", "z"), ("y", "z", "x"), ("z", "x", "y"))
        meta = []
        for g, (col0, w) in enumerate(GROUP_COLS):
            a1, a2, a3 = orders[g]
            b1, b2, b3 = bits[a1], bits[a2], bits[a3]
            k1 = 2048 * b1
            k2l = 1024 * b2
            k3l = k2l + 512 * b3
            meta.append(dict(
                col0=col0, w=w,
                k1=k1, s1=2048 - k1,
                k2l=k2l, s2l=1024 - 1024 * b2,
                k3l=k3l, s3l=k2l + 512 - 512 * b3,
                b3=b3,
                part=(parts[a1], parts[a2], parts[a3]),
            ))

        rdmas = []
        for g, mt in enumerate(meta):
            cols = pl.ds(mt["col0"], mt["w"])
            r = pltpu.make_async_remote_copy(
                src_ref=p_ref.at[pl.ds(mt["s1"], 2048), cols],
                dst_ref=vacc.at[:, cols],
                send_sem=send_sems.at[0, g],
                recv_sem=recv_sems.at[0, g],
                device_id=(mt["part"][0],),
                device_id_type=pl.DeviceIdType.MESH,
            )
            r.start()
            rdmas.append(r)

        tiles = [(g, t) for g in range(len(meta)) for t in range(2048 // CH)]

        def s1_load(ti):
            g, t = tiles[ti]
            mt = meta[g]
            cols = pl.ds(mt["col0"], mt["w"])
            return pltpu.make_async_copy(
                p_ref.at[pl.ds(mt["k1"] + t * CH, CH), cols],
                ptile.at[ti % 2, :, cols], add_sems.at[ti % 2])

        def s1_start(ti):
            g, t = tiles[ti]
            if t == 0:
                rdmas[g].wait()
            s1_load(ti).start()

        s1_start(0)
        for ti in range(len(tiles)):
            if ti + 1 < len(tiles):
                s1_start(ti + 1)
            g, t = tiles[ti]
            mt = meta[g]
            cols = pl.ds(mt["col0"], mt["w"])
            rows = pl.ds(t * CH, CH)
            s1_load(ti).wait()
            vacc[rows, cols] = vacc[rows, cols] + ptile[ti % 2, :, cols]

        for s in (1, 2):
            L = 2048 >> s
            slot = 0 if s == 1 else 1024
            rdmas = []
            for g, mt in enumerate(meta):
                cols = pl.ds(mt["col0"], mt["w"])
                sendb = mt["s2l"] if s == 1 else mt["s3l"]
                r = pltpu.make_async_remote_copy(
                    src_ref=vacc.at[pl.ds(sendb, L), cols],
                    dst_ref=rsbuf.at[pl.ds(slot, L), cols],
                    send_sem=send_sems.at[s, g],
                    recv_sem=recv_sems.at[s, g],
                    device_id=(mt["part"][s],),
                    device_id_type=pl.DeviceIdType.MESH,
                )
                r.start()
                rdmas.append(r)
            stiles = [(g, t) for g in range(len(meta)) for t in range(L // CH)]

            def rs_load(ti, stiles=stiles, slot=slot):
                g, t = stiles[ti]
                mt = meta[g]
                cols = pl.ds(mt["col0"], mt["w"])
                return pltpu.make_async_copy(
                    rsbuf.at[pl.ds(slot + t * CH, CH), cols],
                    ptile.at[ti % 2, :, cols], add_sems.at[ti % 2])

            def rs_start(ti, stiles=stiles, rdmas=rdmas, rs_load=rs_load):
                g, t = stiles[ti]
                if t == 0:
                    rdmas[g].wait()
                rs_load(ti).start()

            rs_start(0)
            for ti in range(len(stiles)):
                if ti + 1 < len(stiles):
                    rs_start(ti + 1)
                g, t = stiles[ti]
                mt = meta[g]
                cols = pl.ds(mt["col0"], mt["w"])
                keepb = mt["k2l"] if s == 1 else mt["k3l"]
                rows = pl.ds(keepb + t * CH, CH)
                rs_load(ti).wait()
                vacc[rows, cols] = vacc[rows, cols] + ptile[ti % 2, :, cols]

        cps = []
        for ci, mt in enumerate(meta):
            cols = pl.ds(mt["col0"], mt["w"])
            cp = pltpu.make_async_copy(
                vacc.at[pl.ds(mt["k3l"], CH), cols],
                out_ref.at[pl.ds(mt["k1"] + mt["k3l"], CH), cols],
                cp_sem.at[ci])
            cp.start()
            cps.append(cp)
        for cp in cps:
            cp.wait()

        def ag_send(g, sid, base, axis_i):
            mt = meta[g]
            cols = pl.ds(mt["col0"], mt["w"])
            rows = pl.ds(base, CH)
            r = pltpu.make_async_remote_copy(
                src_ref=out_ref.at[rows, cols],
                dst_ref=out_ref.at[rows, cols],
                send_sem=ag_send_sems.at[sid, g],
                recv_sem=ag_recv_sems.at[sid, g],
                device_id=(mt["part"][axis_i],),
                device_id_type=pl.DeviceIdType.MESH,
            )
            r.start()
            return r

        def ag_recv_wait(g, sid):
            mt = meta[g]
            cols = pl.ds(mt["col0"], mt["w"])
            rows = pl.ds(0, CH)
            pltpu.make_async_remote_copy(
                src_ref=out_ref.at[rows, cols],
                dst_ref=out_ref.at[rows, cols],
                send_sem=ag_send_sems.at[sid, g],
                recv_sem=ag_recv_sems.at[sid, g],
                device_id=(mt["part"][0],),
                device_id_type=pl.DeviceIdType.MESH,
            ).wait_recv()

        sends = []
        for g, mt in enumerate(meta):
            m0 = mt["k1"] + mt["k3l"]
            sends.append(ag_send(g, 0, m0, 2))
            sends.append(ag_send(g, 1, m0, 1))
            sends.append(ag_send(g, 3, m0, 0))
        for g, mt in enumerate(meta):
            r4 = mt["k1"] + mt["s3l"]
            ag_recv_wait(g, 0)
            sends.append(ag_send(g, 2, r4, 1))
            sends.append(ag_send(g, 4, r4, 0))
        for g, mt in enumerate(meta):
            r5a = mt["k1"] + mt["s2l"] + 512 * mt["b3"]
            ag_recv_wait(g, 1)
            sends.append(ag_send(g, 5, r5a, 0))
        for g, mt in enumerate(meta):
            r5b = mt["k1"] + mt["s2l"] + 512 - 512 * mt["b3"]
            ag_recv_wait(g, 2)
            sends.append(ag_send(g, 6, r5b, 0))
        for g in range(len(meta)):
            for sid in (3, 4, 5, 6):
                ag_recv_wait(g, sid)
        for r in sends:
            r.wait_send()

        @functools.partial(
            pl.run_scoped, second_barrier=pltpu.SemaphoreType.REGULAR)
        def _(second_barrier):
            for nbr in (px, py, pz):
                pl.semaphore_signal(
                    second_barrier, inc=1,
                    device_id=(nbr,), device_id_type=pl.DeviceIdType.MESH,
                )
            pl.semaphore_wait(second_barrier, 3)

    out, _ = pl.pallas_call(
        body,
        out_shape=[
            jax.ShapeDtypeStruct((m, n), jnp.float32),
            jax.ShapeDtypeStruct((1536, n), jnp.float32),
        ],
        in_specs=[pl.BlockSpec(memory_space=pl.ANY)],
        out_specs=[
            pl.BlockSpec(memory_space=pl.ANY),
            pl.BlockSpec(memory_space=pl.ANY),
        ],
        scratch_shapes=[
            pltpu.VMEM((2048, n), jnp.float32),
            pltpu.VMEM((2, CH, n), jnp.float32),
            pltpu.SemaphoreType.DMA((3, 3)),
            pltpu.SemaphoreType.DMA((3, 3)),
            pltpu.SemaphoreType.DMA((2,)),
            pltpu.SemaphoreType.DMA((3,)),
            pltpu.SemaphoreType.DMA((7, 3)),
            pltpu.SemaphoreType.DMA((7, 3)),
        ],
        compiler_params=pltpu.CompilerParams(
            collective_id=0,
            vmem_limit_bytes=63 * 1024 * 1024,
        ),
    )(P)
    return out


# device time: 421284 ns/iter; 1.4921x vs baseline; 1.4708x over previous
import functools

import jax
import jax.numpy as jnp
from jax import lax
from jax.experimental import pallas as pl
from jax.experimental.pallas import tpu as pltpu

N_DEV = 8
CH = 512


def kernel(A, B):
    m = A.shape[0]
    n = B.shape[1]

    GROUP_COLS = ((0, 1408), (1408, 1408), (2816, 1280))

    P = jnp.dot(A, B, preferred_element_type=jnp.bfloat16)

    def body(p_ref, out_ref, rsbuf, agb, vacc, ptile, cvt32,
             send_sems, recv_sems, add_sems, cp_sem,
             ag_send_sems, ag_recv_sems, cvt_ld_sems, cvt_st_sems):
        my = lax.axis_index("i")
        z = my // 4
        jj = my - 4 * z
        y = jj // 2
        p = jj - 2 * y
        x = y + p - 2 * y * p
        px = 4 * z + jj + 1 - 2 * p
        py = 4 * z + 3 - jj
        pz = my + 4 - 8 * z

        barrier_sem = pltpu.get_barrier_semaphore()
        for nbr in (px, py, pz):
            pl.semaphore_signal(
                barrier_sem, inc=1,
                device_id=(nbr,), device_id_type=pl.DeviceIdType.MESH,
            )
        pl.semaphore_wait(barrier_sem, 3)

        bits = {"x": x, "y": y, "z": z}
        parts = {"x": px, "y": py, "z": pz}
        orders = (("x", "y", "z"), ("y", "z", "x"), ("z", "x", "y"))
        meta = []
        for g, (col0, w) in enumerate(GROUP_COLS):
            a1, a2, a3 = orders[g]
            b1, b2, b3 = bits[a1], bits[a2], bits[a3]
            k1 = 2048 * b1
            k2l = 1024 * b2
            k3l = k2l + 512 * b3
            meta.append(dict(
                col0=col0, w=w,
                k1=k1, s1=2048 - k1,
                k2l=k2l, s2l=1024 - 1024 * b2,
                k3l=k3l, s3l=k2l + 512 - 512 * b3,
                b3=b3,
                part=(parts[a1], parts[a2], parts[a3]),
            ))

        rdmas = []
        for g, mt in enumerate(meta):
            cols = pl.ds(mt["col0"], mt["w"])
            r = pltpu.make_async_remote_copy(
                src_ref=p_ref.at[pl.ds(mt["s1"], 2048), cols],
                dst_ref=vacc.at[:, cols],
                send_sem=send_sems.at[0, g],
                recv_sem=recv_sems.at[0, g],
                device_id=(mt["part"][0],),
                device_id_type=pl.DeviceIdType.MESH,
            )
            r.start()
            rdmas.append(r)

        tiles = [(g, t) for g in range(len(meta)) for t in range(2048 // CH)]

        def s1_load(ti):
            g, t = tiles[ti]
            mt = meta[g]
            cols = pl.ds(mt["col0"], mt["w"])
            return pltpu.make_async_copy(
                p_ref.at[pl.ds(mt["k1"] + t * CH, CH), cols],
                ptile.at[ti % 2, :, cols], add_sems.at[ti % 2])

        def s1_start(ti):
            g, t = tiles[ti]
            if t == 0:
                rdmas[g].wait()
            s1_load(ti).start()

        s1_start(0)
        for ti in range(len(tiles)):
            if ti + 1 < len(tiles):
                s1_start(ti + 1)
            g, t = tiles[ti]
            mt = meta[g]
            cols = pl.ds(mt["col0"], mt["w"])
            rows = pl.ds(t * CH, CH)
            s1_load(ti).wait()
            vacc[rows, cols] = vacc[rows, cols] + ptile[ti % 2, :, cols]

        for s in (1, 2):
            L = 2048 >> s
            slot = 0 if s == 1 else 1024
            rdmas = []
            for g, mt in enumerate(meta):
                cols = pl.ds(mt["col0"], mt["w"])
                sendb = mt["s2l"] if s == 1 else mt["s3l"]
                r = pltpu.make_async_remote_copy(
                    src_ref=vacc.at[pl.ds(sendb, L), cols],
                    dst_ref=rsbuf.at[pl.ds(slot, L), cols],
                    send_sem=send_sems.at[s, g],
                    recv_sem=recv_sems.at[s, g],
                    device_id=(mt["part"][s],),
                    device_id_type=pl.DeviceIdType.MESH,
                )
                r.start()
                rdmas.append(r)
            stiles = [(g, t) for g in range(len(meta)) for t in range(L // CH)]

            def rs_load(ti, stiles=stiles, slot=slot):
                g, t = stiles[ti]
                mt = meta[g]
                cols = pl.ds(mt["col0"], mt["w"])
                return pltpu.make_async_copy(
                    rsbuf.at[pl.ds(slot + t * CH, CH), cols],
                    ptile.at[ti % 2, :, cols], add_sems.at[ti % 2])

            def rs_start(ti, stiles=stiles, rdmas=rdmas, rs_load=rs_load):
                g, t = stiles[ti]
                if t == 0:
                    rdmas[g].wait()
                rs_load(ti).start()

            rs_start(0)
            for ti in range(len(stiles)):
                if ti + 1 < len(stiles):
                    rs_start(ti + 1)
                g, t = stiles[ti]
                mt = meta[g]
                cols = pl.ds(mt["col0"], mt["w"])
                keepb = mt["k2l"] if s == 1 else mt["k3l"]
                rows = pl.ds(keepb + t * CH, CH)
                rs_load(ti).wait()
                vacc[rows, cols] = vacc[rows, cols] + ptile[ti % 2, :, cols]

        cps = []
        for ci, mt in enumerate(meta):
            cols = pl.ds(mt["col0"], mt["w"])
            cp = pltpu.make_async_copy(
                vacc.at[pl.ds(mt["k3l"], CH), cols],
                agb.at[pl.ds(mt["k1"] + mt["k3l"], CH), cols],
                cp_sem.at[ci])
            cp.start()
            cps.append(cp)
        for cp in cps:
            cp.wait()

        def ag_send(g, sid, base, axis_i):
            mt = meta[g]
            cols = pl.ds(mt["col0"], mt["w"])
            rows = pl.ds(base, CH)
            r = pltpu.make_async_remote_copy(
                src_ref=agb.at[rows, cols],
                dst_ref=agb.at[rows, cols],
                send_sem=ag_send_sems.at[sid, g],
                recv_sem=ag_recv_sems.at[sid, g],
                device_id=(mt["part"][axis_i],),
                device_id_type=pl.DeviceIdType.MESH,
            )
            r.start()
            return r

        def ag_recv_wait(g, sid):
            mt = meta[g]
            cols = pl.ds(mt["col0"], mt["w"])
            rows = pl.ds(0, CH)
            pltpu.make_async_remote_copy(
                src_ref=agb.at[rows, cols],
                dst_ref=agb.at[rows, cols],
                send_sem=ag_send_sems.at[sid, g],
                recv_sem=ag_recv_sems.at[sid, g],
                device_id=(mt["part"][0],),
                device_id_type=pl.DeviceIdType.MESH,
            ).wait_recv()

        sends = []
        for g, mt in enumerate(meta):
            m0 = mt["k1"] + mt["k3l"]
            sends.append(ag_send(g, 0, m0, 2))
            sends.append(ag_send(g, 1, m0, 1))
            sends.append(ag_send(g, 3, m0, 0))
        for g, mt in enumerate(meta):
            r4 = mt["k1"] + mt["s3l"]
            ag_recv_wait(g, 0)
            sends.append(ag_send(g, 2, r4, 1))
            sends.append(ag_send(g, 4, r4, 0))
        for g, mt in enumerate(meta):
            r5a = mt["k1"] + mt["s2l"] + 512 * mt["b3"]
            ag_recv_wait(g, 1)
            sends.append(ag_send(g, 5, r5a, 0))
        for g, mt in enumerate(meta):
            r5b = mt["k1"] + mt["s2l"] + 512 - 512 * mt["b3"]
            ag_recv_wait(g, 2)
            sends.append(ag_send(g, 6, r5b, 0))
        for g in range(len(meta)):
            for sid in (3, 4, 5, 6):
                ag_recv_wait(g, sid)
        for r in sends:
            r.wait_send()

        NT = m // CH

        def cv_load(t):
            return pltpu.make_async_copy(
                agb.at[pl.ds(t * CH, CH), :], ptile.at[t % 2],
                cvt_ld_sems.at[t % 2])

        def cv_store(t):
            return pltpu.make_async_copy(
                cvt32.at[t % 2], out_ref.at[pl.ds(t * CH, CH), :],
                cvt_st_sems.at[t % 2])

        cv_load(0).start()
        for t in range(NT):
            if t + 1 < NT:
                cv_load(t + 1).start()
            cv_load(t).wait()
            if t >= 2:
                cv_store(t - 2).wait()
            cvt32[t % 2] = ptile[t % 2].astype(jnp.float32)
            cv_store(t).start()
        for t in range(NT - 2, NT):
            cv_store(t).wait()

        @functools.partial(
            pl.run_scoped, second_barrier=pltpu.SemaphoreType.REGULAR)
        def _(second_barrier):
            for nbr in (px, py, pz):
                pl.semaphore_signal(
                    second_barrier, inc=1,
                    device_id=(nbr,), device_id_type=pl.DeviceIdType.MESH,
                )
            pl.semaphore_wait(second_barrier, 3)

    out, _, _ = pl.pallas_call(
        body,
        out_shape=[
            jax.ShapeDtypeStruct((m, n), jnp.float32),
            jax.ShapeDtypeStruct((1536, n), jnp.bfloat16),
            jax.ShapeDtypeStruct((m, n), jnp.bfloat16),
        ],
        in_specs=[pl.BlockSpec(memory_space=pl.ANY)],
        out_specs=[
            pl.BlockSpec(memory_space=pl.ANY),
            pl.BlockSpec(memory_space=pl.ANY),
            pl.BlockSpec(memory_space=pl.ANY),
        ],
        scratch_shapes=[
            pltpu.VMEM((2048, n), jnp.bfloat16),
            pltpu.VMEM((2, CH, n), jnp.bfloat16),
            pltpu.VMEM((2, CH, n), jnp.float32),
            pltpu.SemaphoreType.DMA((3, 3)),
            pltpu.SemaphoreType.DMA((3, 3)),
            pltpu.SemaphoreType.DMA((2,)),
            pltpu.SemaphoreType.DMA((3,)),
            pltpu.SemaphoreType.DMA((7, 3)),
            pltpu.SemaphoreType.DMA((7, 3)),
            pltpu.SemaphoreType.DMA((2,)),
            pltpu.SemaphoreType.DMA((2,)),
        ],
        compiler_params=pltpu.CompilerParams(
            collective_id=0,
            vmem_limit_bytes=63 * 1024 * 1024,
        ),
    )(P)
    return out


# device time: 412452 ns/iter; 1.5241x vs baseline; 1.0214x over previous
import functools

import jax
import jax.numpy as jnp
from jax import lax
from jax.experimental import pallas as pl
from jax.experimental.pallas import tpu as pltpu

N_DEV = 8
CH = 512


def kernel(A, B):
    m = A.shape[0]
    n = B.shape[1]

    GROUP_COLS = ((0, 1408), (1408, 1408), (2816, 1280))

    P = jnp.dot(A, B, preferred_element_type=jnp.bfloat16)

    def body(p_ref, out_ref, rsbuf, agb, vacc, ptile, cvt32,
             send_sems, recv_sems, add_sems, cp_sem,
             ag_send_sems, ag_recv_sems, cvt_ld_sems, cvt_st_sems):
        my = lax.axis_index("i")
        z = my // 4
        jj = my - 4 * z
        y = jj // 2
        p = jj - 2 * y
        x = y + p - 2 * y * p
        px = 4 * z + jj + 1 - 2 * p
        py = 4 * z + 3 - jj
        pz = my + 4 - 8 * z

        barrier_sem = pltpu.get_barrier_semaphore()
        for nbr in (px, py, pz):
            pl.semaphore_signal(
                barrier_sem, inc=1,
                device_id=(nbr,), device_id_type=pl.DeviceIdType.MESH,
            )
        pl.semaphore_wait(barrier_sem, 3)

        bits = {"x": x, "y": y, "z": z}
        parts = {"x": px, "y": py, "z": pz}
        orders = (("x", "y", "z"), ("y", "z", "x"), ("z", "x", "y"))
        meta = []
        for g, (col0, w) in enumerate(GROUP_COLS):
            a1, a2, a3 = orders[g]
            b1, b2, b3 = bits[a1], bits[a2], bits[a3]
            k1 = 2048 * b1
            k2l = 1024 * b2
            k3l = k2l + 512 * b3
            meta.append(dict(
                col0=col0, w=w,
                k1=k1, s1=2048 - k1,
                k2l=k2l, s2l=1024 - 1024 * b2,
                k3l=k3l, s3l=k2l + 512 - 512 * b3,
                b3=b3,
                part=(parts[a1], parts[a2], parts[a3]),
            ))

        rdmas = []
        for g, mt in enumerate(meta):
            cols = pl.ds(mt["col0"], mt["w"])
            r = pltpu.make_async_remote_copy(
                src_ref=p_ref.at[pl.ds(mt["s1"], 2048), cols],
                dst_ref=vacc.at[:, cols],
                send_sem=send_sems.at[0, g],
                recv_sem=recv_sems.at[0, g],
                device_id=(mt["part"][0],),
                device_id_type=pl.DeviceIdType.MESH,
            )
            r.start()
            rdmas.append(r)

        rdmas2 = [None, None, None]
        rdmas3 = [None, None, None]

        def start_stage2(g):
            mt = meta[g]
            cols = pl.ds(mt["col0"], mt["w"])
            r = pltpu.make_async_remote_copy(
                src_ref=vacc.at[pl.ds(mt["s2l"], 1024), cols],
                dst_ref=rsbuf.at[pl.ds(0, 1024), cols],
                send_sem=send_sems.at[1, g],
                recv_sem=recv_sems.at[1, g],
                device_id=(mt["part"][1],),
                device_id_type=pl.DeviceIdType.MESH,
            )
            r.start()
            rdmas2[g] = r

        def start_stage3(g):
            mt = meta[g]
            cols = pl.ds(mt["col0"], mt["w"])
            r = pltpu.make_async_remote_copy(
                src_ref=vacc.at[pl.ds(mt["s3l"], CH), cols],
                dst_ref=rsbuf.at[pl.ds(1024, CH), cols],
                send_sem=send_sems.at[2, g],
                recv_sem=recv_sems.at[2, g],
                device_id=(mt["part"][2],),
                device_id_type=pl.DeviceIdType.MESH,
            )
            r.start()
            rdmas3[g] = r

        tiles1 = []
        for g, mt in enumerate(meta):
            tiles1 += [(g, mt["s2l"]), (g, mt["s2l"] + CH),
                       (g, mt["k2l"]), (g, mt["k2l"] + CH)]

        def s1_load(ti):
            g, base = tiles1[ti]
            mt = meta[g]
            cols = pl.ds(mt["col0"], mt["w"])
            return pltpu.make_async_copy(
                p_ref.at[pl.ds(mt["k1"] + base, CH), cols],
                ptile.at[ti % 2, :, cols], add_sems.at[ti % 2])

        def s1_start(ti):
            g, _ = tiles1[ti]
            if ti % 4 == 0:
                rdmas[g].wait()
            s1_load(ti).start()

        s1_start(0)
        for ti in range(len(tiles1)):
            if ti + 1 < len(tiles1):
                s1_start(ti + 1)
            g, base = tiles1[ti]
            mt = meta[g]
            cols = pl.ds(mt["col0"], mt["w"])
            rows = pl.ds(base, CH)
            s1_load(ti).wait()
            vacc[rows, cols] = vacc[rows, cols] + ptile[ti % 2, :, cols]
            if ti % 4 == 1:
                start_stage2(g)

        tiles2 = []
        for g, mt in enumerate(meta):
            tiles2 += [(g, mt["s3l"]), (g, mt["k3l"])]

        def s2_load(ti):
            g, base = tiles2[ti]
            mt = meta[g]
            cols = pl.ds(mt["col0"], mt["w"])
            return pltpu.make_async_copy(
                rsbuf.at[pl.ds(base - mt["k2l"], CH), cols],
                ptile.at[ti % 2, :, cols], add_sems.at[ti % 2])

        def s2_start(ti):
            g, _ = tiles2[ti]
            if ti % 2 == 0:
                rdmas2[g].wait()
            s2_load(ti).start()

        s2_start(0)
        for ti in range(len(tiles2)):
            if ti + 1 < len(tiles2):
                s2_start(ti + 1)
            g, base = tiles2[ti]
            mt = meta[g]
            cols = pl.ds(mt["col0"], mt["w"])
            rows = pl.ds(base, CH)
            s2_load(ti).wait()
            vacc[rows, cols] = vacc[rows, cols] + ptile[ti % 2, :, cols]
            if ti % 2 == 0:
                start_stage3(g)

        def s3_load(g):
            mt = meta[g]
            cols = pl.ds(mt["col0"], mt["w"])
            return pltpu.make_async_copy(
                rsbuf.at[pl.ds(1024, CH), cols],
                ptile.at[g % 2, :, cols], add_sems.at[g % 2])

        def s3_start(g):
            rdmas3[g].wait()
            s3_load(g).start()

        s3_start(0)
        for g, mt in enumerate(meta):
            if g + 1 < len(meta):
                s3_start(g + 1)
            cols = pl.ds(mt["col0"], mt["w"])
            rows = pl.ds(mt["k3l"], CH)
            s3_load(g).wait()
            vacc[rows, cols] = vacc[rows, cols] + ptile[g % 2, :, cols]

        cps = []
        for ci, mt in enumerate(meta):
            cols = pl.ds(mt["col0"], mt["w"])
            cp = pltpu.make_async_copy(
                vacc.at[pl.ds(mt["k3l"], CH), cols],
                agb.at[pl.ds(mt["k1"] + mt["k3l"], CH), cols],
                cp_sem.at[ci])
            cp.start()
            cps.append(cp)
        for cp in cps:
            cp.wait()

        def ag_send(g, sid, base, axis_i):
            mt = meta[g]
            cols = pl.ds(mt["col0"], mt["w"])
            rows = pl.ds(base, CH)
            r = pltpu.make_async_remote_copy(
                src_ref=agb.at[rows, cols],
                dst_ref=agb.at[rows, cols],
                send_sem=ag_send_sems.at[sid, g],
                recv_sem=ag_recv_sems.at[sid, g],
                device_id=(mt["part"][axis_i],),
                device_id_type=pl.DeviceIdType.MESH,
            )
            r.start()
            return r

        def ag_recv_wait(g, sid):
            mt = meta[g]
            cols = pl.ds(mt["col0"], mt["w"])
            rows = pl.ds(0, CH)
            pltpu.make_async_remote_copy(
                src_ref=agb.at[rows, cols],
                dst_ref=agb.at[rows, cols],
                send_sem=ag_send_sems.at[sid, g],
                recv_sem=ag_recv_sems.at[sid, g],
                device_id=(mt["part"][0],),
                device_id_type=pl.DeviceIdType.MESH,
            ).wait_recv()

        sends = []
        for g, mt in enumerate(meta):
            m0 = mt["k1"] + mt["k3l"]
            sends.append(ag_send(g, 0, m0, 2))
            sends.append(ag_send(g, 1, m0, 1))
            sends.append(ag_send(g, 3, m0, 0))
        for g, mt in enumerate(meta):
            r4 = mt["k1"] + mt["s3l"]
            ag_recv_wait(g, 0)
            sends.append(ag_send(g, 2, r4, 1))
            sends.append(ag_send(g, 4, r4, 0))
        for g, mt in enumerate(meta):
            r5a = mt["k1"] + mt["s2l"] + 512 * mt["b3"]
            ag_recv_wait(g, 1)
            sends.append(ag_send(g, 5, r5a, 0))
        for g, mt in enumerate(meta):
            r5b = mt["k1"] + mt["s2l"] + 512 - 512 * mt["b3"]
            ag_recv_wait(g, 2)
            sends.append(ag_send(g, 6, r5b, 0))
        for g in range(len(meta)):
            for sid in (3, 4, 5, 6):
                ag_recv_wait(g, sid)
        for r in sends:
            r.wait_send()

        NT = m // CH

        def cv_load(t):
            return pltpu.make_async_copy(
                agb.at[pl.ds(t * CH, CH), :], ptile.at[t % 2],
                cvt_ld_sems.at[t % 2])

        def cv_store(t):
            return pltpu.make_async_copy(
                cvt32.at[t % 2], out_ref.at[pl.ds(t * CH, CH), :],
                cvt_st_sems.at[t % 2])

        cv_load(0).start()
        for t in range(NT):
            if t + 1 < NT:
                cv_load(t + 1).start()
            cv_load(t).wait()
            if t >= 2:
                cv_store(t - 2).wait()
            cvt32[t % 2] = ptile[t % 2].astype(jnp.float32)
            cv_store(t).start()
        for t in range(NT - 2, NT):
            cv_store(t).wait()

        @functools.partial(
            pl.run_scoped, second_barrier=pltpu.SemaphoreType.REGULAR)
        def _(second_barrier):
            for nbr in (px, py, pz):
                pl.semaphore_signal(
                    second_barrier, inc=1,
                    device_id=(nbr,), device_id_type=pl.DeviceIdType.MESH,
                )
            pl.semaphore_wait(second_barrier, 3)

    out, _, _ = pl.pallas_call(
        body,
        out_shape=[
            jax.ShapeDtypeStruct((m, n), jnp.float32),
            jax.ShapeDtypeStruct((1536, n), jnp.bfloat16),
            jax.ShapeDtypeStruct((m, n), jnp.bfloat16),
        ],
        in_specs=[pl.BlockSpec(memory_space=pl.ANY)],
        out_specs=[
            pl.BlockSpec(memory_space=pl.ANY),
            pl.BlockSpec(memory_space=pl.ANY),
            pl.BlockSpec(memory_space=pl.ANY),
        ],
        scratch_shapes=[
            pltpu.VMEM((2048, n), jnp.bfloat16),
            pltpu.VMEM((2, CH, n), jnp.bfloat16),
            pltpu.VMEM((2, CH, n), jnp.float32),
            pltpu.SemaphoreType.DMA((3, 3)),
            pltpu.SemaphoreType.DMA((3, 3)),
            pltpu.SemaphoreType.DMA((2,)),
            pltpu.SemaphoreType.DMA((3,)),
            pltpu.SemaphoreType.DMA((7, 3)),
            pltpu.SemaphoreType.DMA((7, 3)),
            pltpu.SemaphoreType.DMA((2,)),
            pltpu.SemaphoreType.DMA((2,)),
        ],
        compiler_params=pltpu.CompilerParams(
            collective_id=0,
            vmem_limit_bytes=63 * 1024 * 1024,
        ),
    )(P)
    return out
